# baseline (speedup 1.0000x reference)
import functools

import jax
import jax.numpy as jnp
from jax import lax
from jax.experimental import pallas as pl
from jax.experimental.pallas import tpu as pltpu
from jax.experimental.pallas import tpu_sc as plsc

NC = 2
NS = 16
NW = NC * NS

B = 16384
D = 64
HID = 32
ACT = 18

LANES = 16

ROWS_W = B // NW
CH = 512
NCH = ROWS_W // CH
NG = CH // LANES


@functools.cache
def _make_sc_gather():
    mesh = plsc.VectorSubcoreMesh(
        core_axis_name="c", subcore_axis_name="s", num_cores=NC, num_subcores=NS
    )

    @functools.partial(
        pl.kernel,
        out_type=jax.ShapeDtypeStruct((NW, NCH, CH, D), jnp.float32),
        mesh=mesh,
        scratch_types=[
            pltpu.VMEM((NCH, CH), jnp.int32),
            pltpu.VMEM((min(2, NCH), CH, D), jnp.float32),
            pltpu.SemaphoreType.DMA,
            pltpu.SemaphoreType.DMA,
            pltpu.SemaphoreType.DMA,
        ],
        compiler_params=pltpu.CompilerParams(needs_layout_passes=False),
    )
    def _sc_gather(idx_hbm, table_hbm, out_hbm, idx_v, x_v, semA, semB, semX):
        wid = lax.axis_index("s") * NC + lax.axis_index("c")
        pltpu.sync_copy(idx_hbm.at[wid], idx_v)
        iota = lax.iota(jnp.int32, LANES)
        sems = (semA, semB)

        def fire(c):
            descs = []
            for g in range(NG):
                vec = idx_v[c, pl.ds(g * LANES, LANES)]
                for r in range(LANES):
                    i = lax.reduce_max(jnp.where(iota == r, vec, -1), axes=(0,))
                    descs.append(
                        pltpu.async_copy(
                            table_hbm.at[i >> 3, i & 7],
                            x_v.at[c % 2, g * LANES + r],
                            sems[c % 2],
                        )
                    )
            return descs

        pend = {}
        for c in range(NCH):
            if c >= 2:
                pltpu.make_async_copy(
                    x_v.at[c % 2], out_hbm.at[wid, c - 2], semX
                ).wait()
            pend[c] = fire(c)
            if c >= 1:
                for cp in pend.pop(c - 1):
                    cp.wait()
                pltpu.async_copy(
                    x_v.at[(c - 1) % 2], out_hbm.at[wid, c - 1], semX
                )
        for cp in pend.pop(NCH - 1):
            cp.wait()
        pltpu.async_copy(x_v.at[(NCH - 1) % 2], out_hbm.at[wid, NCH - 1], semX)
        for c in range(max(0, NCH - 2), NCH):
            pltpu.make_async_copy(
                x_v.at[c % 2], out_hbm.at[wid, c], semX
            ).wait()

    return _sc_gather


def _mlp_body(x_ref, w1_ref, b1_ref, w2_ref, b2_ref, eye_ref, yT_ref, xT_ref):
    x = x_ref[...]
    h = lax.dot_general(
        x, w1_ref[...], (((1,), (1,)), ((), ())),
        preferred_element_type=jnp.float32,
    )
    h = jnp.maximum(h + b1_ref[...], 0.0)
    yT = lax.dot_general(
        w2_ref[...], h, (((1,), (1,)), ((), ())),
        preferred_element_type=jnp.float32,
    )
    yT_ref[...] = yT + b2_ref[...]
    xT_ref[...] = lax.dot_general(
        eye_ref[...], x, (((1,), (1,)), ((), ())),
        preferred_element_type=jnp.float32,
    )


def _mlp(x, W1, b1, W2, b2):
    BB = 4096
    grid = (B // BB,)
    return pl.pallas_call(
        _mlp_body,
        grid=grid,
        in_specs=[
            pl.BlockSpec((BB, D), lambda i: (i, 0)),
            pl.BlockSpec((HID, D), lambda i: (0, 0)),
            pl.BlockSpec((1, HID), lambda i: (0, 0)),
            pl.BlockSpec((ACT, HID), lambda i: (0, 0)),
            pl.BlockSpec((ACT, 1), lambda i: (0, 0)),
            pl.BlockSpec((D, D), lambda i: (0, 0)),
        ],
        out_specs=[
            pl.BlockSpec((ACT, BB), lambda i: (0, i)),
            pl.BlockSpec((D, BB), lambda i: (0, i)),
        ],
        out_shape=[
            jax.ShapeDtypeStruct((ACT, B), jnp.float32),
            jax.ShapeDtypeStruct((D, B), jnp.float32),
        ],
    )(x, W1, b1.reshape(1, HID), W2, b2.reshape(ACT, 1), jnp.eye(D, dtype=jnp.float32))


def kernel(states, table, W1, b1, W2, b2):
    idx = states.reshape(NW, NCH, CH)
    table3 = table.reshape(1000000 // 8, 8, D)
    x = _make_sc_gather()(idx, table3).reshape(B, D)
    yT, xT = _mlp(x, W1, b1, W2, b2)
    return (yT.T, xT.T)


# speedup vs baseline: 1.0029x; 1.0029x over previous
import functools

import jax
import jax.numpy as jnp
from jax import lax
from jax.experimental import pallas as pl
from jax.experimental.pallas import tpu as pltpu
from jax.experimental.pallas import tpu_sc as plsc

NC = 2
NS = 16
NW = NC * NS

B = 16384
D = 64
HID = 32
ACT = 18

LANES = 16

ROWS_W = B // NW
CH = 256
NCH = ROWS_W // CH
NG = CH // LANES


@functools.cache
def _make_sc_gather():
    mesh = plsc.VectorSubcoreMesh(
        core_axis_name="c", subcore_axis_name="s", num_cores=NC, num_subcores=NS
    )

    @functools.partial(
        pl.kernel,
        out_type=jax.ShapeDtypeStruct((NW, NCH, CH, D), jnp.float32),
        mesh=mesh,
        scratch_types=[
            pltpu.VMEM((NCH, CH), jnp.int32),
            pltpu.VMEM((min(2, NCH), CH, D), jnp.float32),
            pltpu.SemaphoreType.DMA,
            pltpu.SemaphoreType.DMA,
            pltpu.SemaphoreType.DMA,
        ],
        compiler_params=pltpu.CompilerParams(needs_layout_passes=False),
    )
    def _sc_gather(idx_hbm, table_hbm, out_hbm, idx_v, x_v, semA, semB, semX):
        wid = lax.axis_index("s") * NC + lax.axis_index("c")
        pltpu.sync_copy(idx_hbm.at[wid], idx_v)
        iota = lax.iota(jnp.int32, LANES)
        sems = (semA, semB)

        def fire(c):
            descs = []
            for g in range(NG):
                vec = idx_v[c, pl.ds(g * LANES, LANES)]
                for r in range(LANES):
                    i = lax.reduce_max(jnp.where(iota == r, vec, -1), axes=(0,))
                    descs.append(
                        pltpu.async_copy(
                            table_hbm.at[i >> 3, i & 7],
                            x_v.at[c % 2, g * LANES + r],
                            sems[c % 2],
                        )
                    )
            return descs

        pend = {}
        for c in range(NCH):
            if c >= 2:
                pltpu.make_async_copy(
                    x_v.at[c % 2], out_hbm.at[wid, c - 2], semX
                ).wait()
            pend[c] = fire(c)
            if c >= 1:
                for cp in pend.pop(c - 1):
                    cp.wait()
                pltpu.async_copy(
                    x_v.at[(c - 1) % 2], out_hbm.at[wid, c - 1], semX
                )
        for cp in pend.pop(NCH - 1):
            cp.wait()
        pltpu.async_copy(x_v.at[(NCH - 1) % 2], out_hbm.at[wid, NCH - 1], semX)
        for c in range(max(0, NCH - 2), NCH):
            pltpu.make_async_copy(
                x_v.at[c % 2], out_hbm.at[wid, c], semX
            ).wait()

    return _sc_gather


def _mlp_body(x_ref, w1_ref, b1_ref, w2_ref, b2_ref, eye_ref, yT_ref, xT_ref):
    x = x_ref[...]
    h = lax.dot_general(
        x, w1_ref[...], (((1,), (1,)), ((), ())),
        preferred_element_type=jnp.float32,
    )
    h = jnp.maximum(h + b1_ref[...], 0.0)
    yT = lax.dot_general(
        w2_ref[...], h, (((1,), (1,)), ((), ())),
        preferred_element_type=jnp.float32,
    )
    yT_ref[...] = yT + b2_ref[...]
    xT_ref[...] = lax.dot_general(
        eye_ref[...], x, (((1,), (1,)), ((), ())),
        preferred_element_type=jnp.float32,
    )


def _mlp(x, W1, b1, W2, b2):
    BB = 4096
    grid = (B // BB,)
    return pl.pallas_call(
        _mlp_body,
        grid=grid,
        in_specs=[
            pl.BlockSpec((BB, D), lambda i: (i, 0)),
            pl.BlockSpec((HID, D), lambda i: (0, 0)),
            pl.BlockSpec((1, HID), lambda i: (0, 0)),
            pl.BlockSpec((ACT, HID), lambda i: (0, 0)),
            pl.BlockSpec((ACT, 1), lambda i: (0, 0)),
            pl.BlockSpec((D, D), lambda i: (0, 0)),
        ],
        out_specs=[
            pl.BlockSpec((ACT, BB), lambda i: (0, i)),
            pl.BlockSpec((D, BB), lambda i: (0, i)),
        ],
        out_shape=[
            jax.ShapeDtypeStruct((ACT, B), jnp.float32),
            jax.ShapeDtypeStruct((D, B), jnp.float32),
        ],
    )(x, W1, b1.reshape(1, HID), W2, b2.reshape(ACT, 1), jnp.eye(D, dtype=jnp.float32))


def kernel(states, table, W1, b1, W2, b2):
    idx = states.reshape(NW, NCH, CH)
    table3 = table.reshape(1000000 // 8, 8, D)
    x = _make_sc_gather()(idx, table3).reshape(B, D)
    yT, xT = _mlp(x, W1, b1, W2, b2)
    return (yT.T, xT.T)
